# baseline (speedup 1.0000x reference)
import jax
import jax.numpy as jnp
from jax import lax
from jax.experimental import pallas as pl

_N = 10000
_MT = 400
_MT_IN = 1000
_NSEG = 64
_PAGES = 5
_QSCALE = 254.0


def _proj_kernel(x_ref, w_ref, o_ref):
    o_ref[...] = jnp.dot(x_ref[...], w_ref[...],
                         preferred_element_type=jnp.float32
                         ).astype(jnp.bfloat16)


def _layer1_kernel(adj_ref, h_ref, b_ref, w_ref, o_ref, aq_ref, cs_ref):
    i = pl.program_id(0)
    a = adj_ref[...]
    y = jnp.dot(a.astype(jnp.bfloat16), h_ref[...],
                preferred_element_type=jnp.float32)
    y = jnp.maximum(y + b_ref[...], 0.0)
    h_next = jnp.dot(y, w_ref[...], preferred_element_type=jnp.float32)
    o_ref[...] = h_next.astype(jnp.bfloat16)
    aq_ref[...] = jnp.round((a - 0.5) * _QSCALE).astype(jnp.int8)[None]

    @pl.when(i == 0)
    def _init():
        cs_ref[...] = jnp.zeros_like(cs_ref)

    cs_ref[...] += jnp.sum(h_next, axis=0, keepdims=True)


def _layer_kernel(aq_ref, h_ref, hcs_ref, b_ref, w_ref, o_ref, cs_ref):
    i = pl.program_id(0)
    cs = jnp.zeros_like(cs_ref)
    for j in range(_PAGES):
        a = aq_ref[j].astype(jnp.bfloat16)
        y = jnp.dot(a, h_ref[...], preferred_element_type=jnp.float32)
        y = y * (1.0 / _QSCALE) + 0.5 * hcs_ref[...] + b_ref[...]
        y = jnp.maximum(y, 0.0)
        h_next = jnp.dot(y, w_ref[...], preferred_element_type=jnp.float32)
        o_ref[pl.ds(j * _MT, _MT), :] = h_next.astype(jnp.bfloat16)
        cs = cs + jnp.sum(h_next, axis=0, keepdims=True)

    @pl.when(i == 0)
    def _init():
        cs_ref[...] = jnp.zeros_like(cs_ref)

    cs_ref[...] += cs


def _final_kernel(aq_ref, h_ref, hcs_ref, b3_ref, w4_ref, b4_ref, idx_ref,
                  o_ref):
    i = pl.program_id(0)
    nsteps = pl.num_programs(0)
    contrib = jnp.zeros((_NSEG, _NSEG), jnp.float32)
    for j in range(_PAGES):
        a = aq_ref[j].astype(jnp.bfloat16)
        y = jnp.dot(a, h_ref[...], preferred_element_type=jnp.float32)
        y = y * (1.0 / _QSCALE) + 0.5 * hcs_ref[...] + b3_ref[...]
        y = jnp.maximum(y, 0.0)
        y = jnp.dot(y, w4_ref[...], preferred_element_type=jnp.float32)
        y = y + b4_ref[...]
        ids = idx_ref[0, :, pl.ds(j * _MT, _MT)]
        rows = lax.broadcasted_iota(jnp.int32, (_NSEG, _MT), 0)
        onehot = (rows == ids).astype(jnp.float32)
        contrib = contrib + jnp.dot(onehot, y,
                                    preferred_element_type=jnp.float32)

    @pl.when(i == 0)
    def _init():
        o_ref[...] = jnp.zeros_like(o_ref)

    o_ref[...] += contrib

    @pl.when(i == nsteps - 1)
    def _logsoftmax():
        acc = o_ref[...]
        mx = jnp.max(acc, axis=1, keepdims=True)
        lse = jnp.log(jnp.sum(jnp.exp(acc - mx), axis=1, keepdims=True)) + mx
        o_ref[...] = acc - lse


def _project(x, w):
    d_in, d_out = w.shape
    return pl.pallas_call(
        _proj_kernel,
        grid=(_N // _MT_IN,),
        in_specs=[pl.BlockSpec((_MT_IN, d_in), lambda i: (i, 0)),
                  pl.BlockSpec((d_in, d_out), lambda i: (0, 0))],
        out_specs=pl.BlockSpec((_MT_IN, d_out), lambda i: (i, 0)),
        out_shape=jax.ShapeDtypeStruct((_N, d_out), jnp.bfloat16),
    )(x, w)


def _gcn_layer1(adj, h, b, w):
    d = h.shape[1]
    d_out = w.shape[1]
    nm = _N // _MT
    return pl.pallas_call(
        _layer1_kernel,
        grid=(nm,),
        in_specs=[pl.BlockSpec((_MT, _N), lambda i: (i, 0)),
                  pl.BlockSpec((_N, d), lambda i: (0, 0)),
                  pl.BlockSpec((1, d), lambda i: (0, 0)),
                  pl.BlockSpec((d, d_out), lambda i: (0, 0))],
        out_specs=[pl.BlockSpec((_MT, d_out), lambda i: (i, 0)),
                   pl.BlockSpec((1, _MT, _N), lambda i: (i, 0, 0)),
                   pl.BlockSpec((1, d_out), lambda i: (0, 0))],
        out_shape=[jax.ShapeDtypeStruct((_N, d_out), jnp.bfloat16),
                   jax.ShapeDtypeStruct((nm, _MT, _N), jnp.int8),
                   jax.ShapeDtypeStruct((1, d_out), jnp.float32)],
    )(adj, h, b.reshape(1, d), w)


def _gcn_layer(aq, h, hcs, b, w):
    d = h.shape[1]
    d_out = w.shape[1]
    ng = _N // (_MT * _PAGES)
    return pl.pallas_call(
        _layer_kernel,
        grid=(ng,),
        in_specs=[pl.BlockSpec((_PAGES, _MT, _N), lambda i: (i, 0, 0)),
                  pl.BlockSpec((_N, d), lambda i: (0, 0)),
                  pl.BlockSpec((1, d), lambda i: (0, 0)),
                  pl.BlockSpec((1, d), lambda i: (0, 0)),
                  pl.BlockSpec((d, d_out), lambda i: (0, 0))],
        out_specs=[pl.BlockSpec((_MT * _PAGES, d_out), lambda i: (i, 0)),
                   pl.BlockSpec((1, d_out), lambda i: (0, 0))],
        out_shape=[jax.ShapeDtypeStruct((_N, d_out), jnp.bfloat16),
                   jax.ShapeDtypeStruct((1, d_out), jnp.float32)],
    )(aq, h, hcs, b.reshape(1, d), w)


def _final(aq, h, hcs, b3, w4, b4, idx):
    ng = _N // (_MT * _PAGES)
    d = h.shape[1]
    idx3 = idx.astype(jnp.int32).reshape(ng, 1, _MT * _PAGES)
    return pl.pallas_call(
        _final_kernel,
        grid=(ng,),
        in_specs=[pl.BlockSpec((_PAGES, _MT, _N), lambda i: (i, 0, 0)),
                  pl.BlockSpec((_N, d), lambda i: (0, 0)),
                  pl.BlockSpec((1, d), lambda i: (0, 0)),
                  pl.BlockSpec((1, d), lambda i: (0, 0)),
                  pl.BlockSpec((d, _NSEG), lambda i: (0, 0)),
                  pl.BlockSpec((1, _NSEG), lambda i: (0, 0)),
                  pl.BlockSpec((1, 1, _MT * _PAGES), lambda i: (i, 0, 0))],
        out_specs=pl.BlockSpec((_NSEG, _NSEG), lambda i: (0, 0)),
        out_shape=jax.ShapeDtypeStruct((_NSEG, _NSEG), jnp.float32),
    )(aq, h, hcs, b3.reshape(1, d), w4, b4.reshape(1, _NSEG), idx3)


def kernel(x_in, adj, idx, W1, b1, W2, b2, W3, b3, W4, b4):
    h1 = _project(x_in, W1)
    h2, aq, cs2 = _gcn_layer1(adj, h1, b1, W2)
    h3, cs3 = _gcn_layer(aq, h2, cs2, b2, W3)
    return _final(aq, h3, cs3, b3, W4, b4, idx)


# speedup vs baseline: 1.2743x; 1.2743x over previous
import jax
import jax.numpy as jnp
from jax import lax
from jax.experimental import pallas as pl

_N = 10000
_MT = 400
_MT_IN = 1000
_NSEG = 64
_QSCALE = 254.0


def _proj_kernel(x_ref, w_ref, o_ref):
    o_ref[...] = jnp.dot(x_ref[...], w_ref[...],
                         preferred_element_type=jnp.float32
                         ).astype(jnp.bfloat16)


def _layer1_kernel(adj_ref, h_ref, b_ref, w_ref, o_ref, aq_ref, cs_ref):
    i = pl.program_id(0)
    a = adj_ref[...]
    y = jnp.dot(a.astype(jnp.bfloat16), h_ref[...],
                preferred_element_type=jnp.float32)
    y = jnp.maximum(y + b_ref[...], 0.0)
    h_next = jnp.dot(y, w_ref[...], preferred_element_type=jnp.float32)
    o_ref[...] = h_next.astype(jnp.bfloat16)
    aq_ref[...] = jnp.round((a - 0.5) * _QSCALE).astype(jnp.int8)[None]

    @pl.when(i == 0)
    def _init():
        cs_ref[...] = jnp.zeros_like(cs_ref)

    cs_ref[...] += jnp.sum(h_next, axis=0, keepdims=True)


def _layer_kernel(aq_ref, h_ref, hcs_ref, b_ref, w_ref, o_ref, cs_ref):
    i = pl.program_id(0)
    a = aq_ref[0].astype(jnp.bfloat16)
    y = jnp.dot(a, h_ref[...], preferred_element_type=jnp.float32)
    y = y * (1.0 / _QSCALE) + 0.5 * hcs_ref[...] + b_ref[...]
    y = jnp.maximum(y, 0.0)
    h_next = jnp.dot(y, w_ref[...], preferred_element_type=jnp.float32)
    o_ref[...] = h_next.astype(jnp.bfloat16)

    @pl.when(i == 0)
    def _init():
        cs_ref[...] = jnp.zeros_like(cs_ref)

    cs_ref[...] += jnp.sum(h_next, axis=0, keepdims=True)


def _final_kernel(aq_ref, h_ref, hcs_ref, b3_ref, w4_ref, b4_ref, idx_ref,
                  o_ref):
    i = pl.program_id(0)
    nsteps = pl.num_programs(0)
    a = aq_ref[0].astype(jnp.bfloat16)
    y = jnp.dot(a, h_ref[...], preferred_element_type=jnp.float32)
    y = y * (1.0 / _QSCALE) + 0.5 * hcs_ref[...] + b3_ref[...]
    y = jnp.maximum(y, 0.0)
    y = jnp.dot(y, w4_ref[...], preferred_element_type=jnp.float32)
    y = y + b4_ref[...]
    ids = idx_ref[0]
    rows = lax.broadcasted_iota(jnp.int32, (_NSEG, _MT), 0)
    onehot = (rows == ids).astype(jnp.float32)
    contrib = jnp.dot(onehot, y, preferred_element_type=jnp.float32)

    @pl.when(i == 0)
    def _init():
        o_ref[...] = jnp.zeros_like(o_ref)

    o_ref[...] += contrib

    @pl.when(i == nsteps - 1)
    def _logsoftmax():
        acc = o_ref[...]
        mx = jnp.max(acc, axis=1, keepdims=True)
        lse = jnp.log(jnp.sum(jnp.exp(acc - mx), axis=1, keepdims=True)) + mx
        o_ref[...] = acc - lse


def _project(x, w):
    d_in, d_out = w.shape
    return pl.pallas_call(
        _proj_kernel,
        grid=(_N // _MT_IN,),
        in_specs=[pl.BlockSpec((_MT_IN, d_in), lambda i: (i, 0)),
                  pl.BlockSpec((d_in, d_out), lambda i: (0, 0))],
        out_specs=pl.BlockSpec((_MT_IN, d_out), lambda i: (i, 0)),
        out_shape=jax.ShapeDtypeStruct((_N, d_out), jnp.bfloat16),
    )(x, w)


def _gcn_layer1(adj, h, b, w):
    d = h.shape[1]
    d_out = w.shape[1]
    nm = _N // _MT
    return pl.pallas_call(
        _layer1_kernel,
        grid=(nm,),
        in_specs=[pl.BlockSpec((_MT, _N), lambda i: (i, 0)),
                  pl.BlockSpec((_N, d), lambda i: (0, 0)),
                  pl.BlockSpec((1, d), lambda i: (0, 0)),
                  pl.BlockSpec((d, d_out), lambda i: (0, 0))],
        out_specs=[pl.BlockSpec((_MT, d_out), lambda i: (i, 0)),
                   pl.BlockSpec((1, _MT, _N), lambda i: (i, 0, 0)),
                   pl.BlockSpec((1, d_out), lambda i: (0, 0))],
        out_shape=[jax.ShapeDtypeStruct((_N, d_out), jnp.bfloat16),
                   jax.ShapeDtypeStruct((nm, _MT, _N), jnp.int8),
                   jax.ShapeDtypeStruct((1, d_out), jnp.float32)],
    )(adj, h, b.reshape(1, d), w)


def _gcn_layer(aq, h, hcs, b, w):
    d = h.shape[1]
    d_out = w.shape[1]
    nm = _N // _MT
    return pl.pallas_call(
        _layer_kernel,
        grid=(nm,),
        in_specs=[pl.BlockSpec((1, _MT, _N), lambda i: (i, 0, 0)),
                  pl.BlockSpec((_N, d), lambda i: (0, 0)),
                  pl.BlockSpec((1, d), lambda i: (0, 0)),
                  pl.BlockSpec((1, d), lambda i: (0, 0)),
                  pl.BlockSpec((d, d_out), lambda i: (0, 0))],
        out_specs=[pl.BlockSpec((_MT, d_out), lambda i: (i, 0)),
                   pl.BlockSpec((1, d_out), lambda i: (0, 0))],
        out_shape=[jax.ShapeDtypeStruct((_N, d_out), jnp.bfloat16),
                   jax.ShapeDtypeStruct((1, d_out), jnp.float32)],
    )(aq, h, hcs, b.reshape(1, d), w)


def _final(aq, h, hcs, b3, w4, b4, idx):
    nm = _N // _MT
    d = h.shape[1]
    idx3 = idx.astype(jnp.int32).reshape(nm, 1, _MT)
    return pl.pallas_call(
        _final_kernel,
        grid=(nm,),
        in_specs=[pl.BlockSpec((1, _MT, _N), lambda i: (i, 0, 0)),
                  pl.BlockSpec((_N, d), lambda i: (0, 0)),
                  pl.BlockSpec((1, d), lambda i: (0, 0)),
                  pl.BlockSpec((1, d), lambda i: (0, 0)),
                  pl.BlockSpec((d, _NSEG), lambda i: (0, 0)),
                  pl.BlockSpec((1, _NSEG), lambda i: (0, 0)),
                  pl.BlockSpec((1, 1, _MT), lambda i: (i, 0, 0))],
        out_specs=pl.BlockSpec((_NSEG, _NSEG), lambda i: (0, 0)),
        out_shape=jax.ShapeDtypeStruct((_NSEG, _NSEG), jnp.float32),
    )(aq, h, hcs, b3.reshape(1, d), w4, b4.reshape(1, _NSEG), idx3)


def kernel(x_in, adj, idx, W1, b1, W2, b2, W3, b3, W4, b4):
    h1 = _project(x_in, W1)
    h2, aq, cs2 = _gcn_layer1(adj, h1, b1, W2)
    h3, cs3 = _gcn_layer(aq, h2, cs2, b2, W3)
    return _final(aq, h3, cs3, b3, W4, b4, idx)
